# Initial kernel scaffold; baseline (speedup 1.0000x reference)
import jax
import jax.numpy as jnp
from jax.experimental import pallas as pl


def kernel(x, edge_index, edge_label_index, W1l, b1l, W1r, W2l, b2l, W2r, dW1, db1, dW2, db2):
    raise NotImplementedError("write your pallas kernel here")


# speedup vs baseline: 3.2769x; 3.2769x over previous
import functools

import jax
import jax.numpy as jnp
from jax import lax
from jax.experimental import pallas as pl
from jax.experimental.pallas import tpu as pltpu
from jax.experimental.pallas import tpu_sc as plsc

NC, NS, LANES = 2, 16, 16
NW = NC * NS


def _chunk_size(per_worker):
    for k in range(128, 0, -8):
        if per_worker % k == 0:
            return k
    raise ValueError(per_worker)


def _divisor_leq(x, cap):
    for c in range(cap, 0, -1):
        if x % c == 0:
            return c
    return 1


def _pad_rows(n):
    q = 8 * NS
    return ((n + q - 1) // q) * q


@functools.lru_cache(maxsize=None)
def _make_seg_sum(n, dw, e):
    e_per_w = e // NW
    K = _chunk_size(e_per_w)
    n_chunks = e_per_w // K
    n_pad = _pad_rows(n)
    rpt = n_pad // NS
    crows = _divisor_leq(rpt, 64)
    n_copy = rpt // crows
    mesh = plsc.VectorSubcoreMesh(
        core_axis_name="c", subcore_axis_name="s", num_cores=NC, num_subcores=NS
    )
    scratch = [
        pltpu.VMEM((K,), jnp.int32),
        pltpu.VMEM((K,), jnp.int32),
        pltpu.VMEM((K, dw), jnp.float32),
        pltpu.VMEM((crows, dw), jnp.float32),
        pltpu.VMEM_SHARED((n_pad, dw), jnp.float32),
        pltpu.SemaphoreType.DMA,
    ]

    def body(h_hbm, src_hbm, dst_hbm, zrow_hbm, out_hbm,
             src_v, dst_v, rows_v, zbuf, acc_sh, sem):
        cid = lax.axis_index("c")
        sid = lax.axis_index("s")
        wid = sid * NC + cid
        row0 = sid * rpt

        pltpu.sync_copy(zrow_hbm, zbuf)

        @pl.loop(0, n_copy)
        def _(t):
            pltpu.sync_copy(zbuf, acc_sh.at[pl.ds(row0 + t * crows, crows)])

        plsc.subcore_barrier()

        ebase = wid * e_per_w

        @pl.loop(0, n_chunks)
        def _(i):
            off = ebase + i * K
            pltpu.sync_copy(src_hbm.at[pl.ds(off, K)], src_v)
            pltpu.sync_copy(dst_hbm.at[pl.ds(off, K)], dst_v)
            pltpu.async_copy(h_hbm.at[src_v], rows_v, sem).wait()
            pltpu.sync_copy(rows_v, acc_sh.at[dst_v], add=True)

        plsc.subcore_barrier()

        hb = cid * n_pad

        @pl.loop(0, n_copy)
        def _(t):
            o = row0 + t * crows
            pltpu.sync_copy(acc_sh.at[pl.ds(o, crows)], zbuf)
            pltpu.sync_copy(zbuf, out_hbm.at[pl.ds(hb + o, crows)])

    return pl.kernel(
        body,
        out_type=jax.ShapeDtypeStruct((NC * n_pad, dw), jnp.float32),
        mesh=mesh,
        scratch_types=scratch,
    )


def _seg_sum_zeros(n, dw):
    crows = _divisor_leq(_pad_rows(n) // NS, 64)
    return jnp.zeros((crows, dw), jnp.float32)


@functools.lru_cache(maxsize=None)
def _make_counts(n, d, e):
    e_per_w = e // NW
    K = _chunk_size(e_per_w)
    n_chunks = e_per_w // K
    n_pad = _pad_rows(n)
    rpt = n_pad // NS
    crows = _divisor_leq(rpt, 64)
    n_copy = rpt // crows
    mesh = plsc.VectorSubcoreMesh(
        core_axis_name="c", subcore_axis_name="s", num_cores=NC, num_subcores=NS
    )
    scratch = [
        pltpu.VMEM((K,), jnp.int32),
        pltpu.VMEM((K, d), jnp.float32),
        pltpu.VMEM((crows, d), jnp.float32),
        pltpu.VMEM_SHARED((n_pad, d), jnp.float32),
    ]

    def body(dst_hbm, ones_hbm, zrow_hbm, out_hbm,
             dst_v, ones_v, zbuf, acc_sh):
        cid = lax.axis_index("c")
        sid = lax.axis_index("s")
        wid = sid * NC + cid
        row0 = sid * rpt

        pltpu.sync_copy(zrow_hbm, zbuf)
        pltpu.sync_copy(ones_hbm, ones_v)

        @pl.loop(0, n_copy)
        def _(t):
            pltpu.sync_copy(zbuf, acc_sh.at[pl.ds(row0 + t * crows, crows)])

        plsc.subcore_barrier()

        ebase = wid * e_per_w

        @pl.loop(0, n_chunks)
        def _(i):
            off = ebase + i * K
            pltpu.sync_copy(dst_hbm.at[pl.ds(off, K)], dst_v)
            pltpu.sync_copy(ones_v, acc_sh.at[dst_v], add=True)

        plsc.subcore_barrier()

        hb = cid * n_pad

        @pl.loop(0, n_copy)
        def _(t):
            o = row0 + t * crows
            pltpu.sync_copy(acc_sh.at[pl.ds(o, crows)], zbuf)
            pltpu.sync_copy(zbuf, out_hbm.at[pl.ds(hb + o, crows)])

    return pl.kernel(
        body,
        out_type=jax.ShapeDtypeStruct((NC * n_pad, d), jnp.float32),
        mesh=mesh,
        scratch_types=scratch,
    )


@functools.lru_cache(maxsize=None)
def _make_decoder(n, h, el):
    el_per_w = el // NW
    K = _chunk_size(el_per_w)
    n_chunks = el_per_w // K
    JV = h // LANES
    mesh = plsc.VectorSubcoreMesh(
        core_axis_name="c", subcore_axis_name="s", num_cores=NC, num_subcores=NS
    )
    scratch = [
        pltpu.VMEM((K,), jnp.int32),
        pltpu.VMEM((K,), jnp.int32),
        pltpu.VMEM((K, h), jnp.float32),
        pltpu.VMEM((K, h), jnp.float32),
        pltpu.VMEM((K, LANES), jnp.float32),
        pltpu.VMEM((h,), jnp.float32),
        pltpu.VMEM((LANES,), jnp.float32),
        pltpu.SemaphoreType.DMA,
    ]

    def body(za_hbm, zb_hbm, row_hbm, col_hbm, w2_hbm, z16_hbm, out_hbm,
             ri_v, ci_v, a_v, b_v, out_v, w2_v, z16_v, sem):
        cid = lax.axis_index("c")
        sid = lax.axis_index("s")
        wid = sid * NC + cid
        ebase = wid * el_per_w
        pltpu.sync_copy(w2_hbm, w2_v)
        pltpu.sync_copy(z16_hbm, z16_v)
        w2s = [w2_v[pl.ds(j * LANES, LANES)] for j in range(JV)]
        zv = z16_v[:]

        @pl.loop(0, n_chunks)
        def _(i):
            off = ebase + i * K
            pltpu.sync_copy(row_hbm.at[pl.ds(off, K)], ri_v)
            pltpu.sync_copy(col_hbm.at[pl.ds(off, K)], ci_v)
            ca = pltpu.async_copy(za_hbm.at[ri_v], a_v, sem)
            cb = pltpu.async_copy(zb_hbm.at[ci_v], b_v, sem)
            ca.wait()
            cb.wait()

            @pl.loop(0, K)
            def _(e):
                acc = None
                for j in range(JV):
                    s = pl.ds(j * LANES, LANES)
                    t = jnp.maximum(a_v[e, s] + b_v[e, s], zv)
                    acc = t * w2s[j] if acc is None else acc + t * w2s[j]
                out_v[e, :] = acc

            pltpu.sync_copy(out_v, out_hbm.at[pl.ds(off, K)])

    return pl.kernel(
        body,
        out_type=jax.ShapeDtypeStruct((el, LANES), jnp.float32),
        mesh=mesh,
        scratch_types=scratch,
    )


def _tc_layer1(x, p0, p1, c0, c1, w1l, w1r, b1l):
    n, d = x.shape
    h = w1l.shape[1]
    rb = 1000
    grid = n // rb

    def body(x_ref, p0_ref, p1_ref, c0_ref, c1_ref, wl_ref, wr_ref, b_ref,
             o_ref):
        cnt_col = c0_ref[:, 0:1] + c1_ref[:, 0:1]
        inv = 1.0 / jnp.maximum(cnt_col, 1.0)
        mean = (p0_ref[...] + p1_ref[...]) * inv
        z = (
            jnp.dot(mean, wl_ref[...], preferred_element_type=jnp.float32)
            + jnp.dot(x_ref[...], wr_ref[...], preferred_element_type=jnp.float32)
            + b_ref[...]
        )
        o_ref[...] = jnp.maximum(z, 0.0)

    return pl.pallas_call(
        body,
        grid=(grid,),
        in_specs=[
            pl.BlockSpec((rb, d), lambda i: (i, 0)),
            pl.BlockSpec((rb, d), lambda i: (i, 0)),
            pl.BlockSpec((rb, d), lambda i: (i, 0)),
            pl.BlockSpec((rb, LANES), lambda i: (i, 0)),
            pl.BlockSpec((rb, LANES), lambda i: (i, 0)),
            pl.BlockSpec((d, h), lambda i: (0, 0)),
            pl.BlockSpec((d, h), lambda i: (0, 0)),
            pl.BlockSpec((1, h), lambda i: (0, 0)),
        ],
        out_specs=pl.BlockSpec((rb, h), lambda i: (i, 0)),
        out_shape=jax.ShapeDtypeStruct((n, h), jnp.float32),
    )(x, p0, p1, c0, c1, w1l, w1r, b1l)


def _tc_layer2_dec(z1, q0, q1, c0, c1, w2l, w2r, b2l, dw1a, dw1b, db1):
    n, h = z1.shape
    rb = 1000
    grid = n // rb

    def body(z1_ref, p0_ref, p1_ref, c0_ref, c1_ref, wl_ref, wr_ref, b_ref,
             da_ref, dbm_ref, db1_ref, oa_ref, ob_ref):
        cnt_col = c0_ref[:, 0:1] + c1_ref[:, 0:1]
        inv = 1.0 / jnp.maximum(cnt_col, 1.0)
        mean = (p0_ref[...] + p1_ref[...]) * inv
        z2 = (
            jnp.dot(mean, wl_ref[...], preferred_element_type=jnp.float32)
            + jnp.dot(z1_ref[...], wr_ref[...], preferred_element_type=jnp.float32)
            + b_ref[...]
        )
        oa_ref[...] = (
            jnp.dot(z2, da_ref[...], preferred_element_type=jnp.float32)
            + db1_ref[...]
        )
        ob_ref[...] = jnp.dot(z2, dbm_ref[...], preferred_element_type=jnp.float32)

    return pl.pallas_call(
        body,
        grid=(grid,),
        in_specs=[
            pl.BlockSpec((rb, h), lambda i: (i, 0)),
            pl.BlockSpec((rb, h), lambda i: (i, 0)),
            pl.BlockSpec((rb, h), lambda i: (i, 0)),
            pl.BlockSpec((rb, LANES), lambda i: (i, 0)),
            pl.BlockSpec((rb, LANES), lambda i: (i, 0)),
            pl.BlockSpec((h, h), lambda i: (0, 0)),
            pl.BlockSpec((h, h), lambda i: (0, 0)),
            pl.BlockSpec((1, h), lambda i: (0, 0)),
            pl.BlockSpec((h, h), lambda i: (0, 0)),
            pl.BlockSpec((h, h), lambda i: (0, 0)),
            pl.BlockSpec((1, h), lambda i: (0, 0)),
        ],
        out_specs=[
            pl.BlockSpec((rb, h), lambda i: (i, 0)),
            pl.BlockSpec((rb, h), lambda i: (i, 0)),
        ],
        out_shape=[
            jax.ShapeDtypeStruct((n, h), jnp.float32),
            jax.ShapeDtypeStruct((n, h), jnp.float32),
        ],
    )(z1, q0, q1, c0, c1, w2l, w2r, b2l, dw1a, dw1b, db1)


def _tc_lane_reduce(inter, db2):
    el = inter.shape[0]
    rb = 4000
    grid = el // rb

    def body(x_ref, b_ref, o_ref):
        o_ref[...] = jnp.sum(x_ref[...], axis=1, keepdims=True) + b_ref[...]

    return pl.pallas_call(
        body,
        grid=(grid,),
        in_specs=[
            pl.BlockSpec((rb, LANES), lambda i: (i, 0)),
            pl.BlockSpec((1, 1), lambda i: (0, 0)),
        ],
        out_specs=pl.BlockSpec((rb, 1), lambda i: (i, 0)),
        out_shape=jax.ShapeDtypeStruct((el, 1), jnp.float32),
    )(inter, db2)


def kernel(x, edge_index, edge_label_index, W1l, b1l, W1r, W2l, b2l, W2r,
           dW1, db1, dW2, db2):
    n, d = x.shape
    h = W1l.shape[1]
    e = edge_index.shape[1]
    el = edge_label_index.shape[1]
    n_pad = _pad_rows(n)
    src = edge_index[0].astype(jnp.int32)
    dst = edge_index[1].astype(jnp.int32)
    row = edge_label_index[0].astype(jnp.int32)
    col = edge_label_index[1].astype(jnp.int32)

    agg1 = _make_seg_sum(n, d, e)(x, src, dst, _seg_sum_zeros(n, d))
    p0, p1 = agg1[:n], agg1[n_pad:n_pad + n]
    K = _chunk_size(e // NW)
    cntf = _make_counts(n, d, e)(
        dst, jnp.ones((K, d), jnp.float32), _seg_sum_zeros(n, d))
    c0, c1 = cntf[:n, :LANES], cntf[n_pad:n_pad + n, :LANES]
    z1 = _tc_layer1(x, p0, p1, c0, c1, W1l, W1r, b1l.reshape(1, h))

    agg2 = _make_seg_sum(n, h, e)(z1, src, dst, _seg_sum_zeros(n, h))
    q0, q1 = agg2[:n], agg2[n_pad:n_pad + n]
    zA, zB = _tc_layer2_dec(
        z1, q0, q1, c0, c1, W2l, W2r, b2l.reshape(1, h),
        dW1[:h], dW1[h:], db1.reshape(1, h),
    )

    inter = _make_decoder(n, h, el)(
        zA, zB, row, col, dW2.reshape(h), jnp.zeros((LANES,), jnp.float32))
    out = _tc_lane_reduce(inter, db2.reshape(1, 1))
    return out.reshape(el)
